# Initial kernel scaffold; baseline (speedup 1.0000x reference)
import jax
import jax.numpy as jnp
from jax.experimental import pallas as pl


def kernel(x, g, b_ln, W_qkv, W_out, b_out):
    raise NotImplementedError("write your pallas kernel here")


# speedup vs baseline: 1.3422x; 1.3422x over previous
import jax
import jax.numpy as jnp
from jax.experimental import pallas as pl

HEADS = 8
DIM_HEAD = 32
DIM = 768
INNER = HEADS * DIM_HEAD
EPS = 1e-5


def _ln_qkv_kernel(x_ref, g_ref, bln_ref, wqkv_ref, qkv_ref):
    x = x_ref[0]
    mean = jnp.mean(x, axis=0, keepdims=True)
    var = jnp.mean((x - mean) ** 2, axis=0, keepdims=True)
    xn = (x - mean) * jax.lax.rsqrt(var + EPS)
    xn = xn * g_ref[...].reshape(DIM, 1) + bln_ref[...].reshape(DIM, 1)
    qkv_ref[0] = jnp.dot(wqkv_ref[...], xn, preferred_element_type=jnp.float32)


def _attn_kernel(q_ref, k_ref, v_ref, o_ref):
    q = q_ref[0]
    k = k_ref[0]
    v = v_ref[0]
    qn = q * jax.lax.rsqrt(jnp.maximum(jnp.sum(q * q, axis=-1, keepdims=True),
                                       1e-24))
    kn = k * jax.lax.rsqrt(jnp.maximum(jnp.sum(k * k, axis=-1, keepdims=True),
                                       1e-24))
    sim = jnp.dot(qn, kn.T, preferred_element_type=jnp.float32)
    m = jnp.max(sim, axis=-1, keepdims=True)
    e = jnp.exp(sim - m)
    p = e / jnp.sum(e, axis=-1, keepdims=True)
    o_ref[0] = jnp.dot(p, v, preferred_element_type=jnp.float32)


def _out_proj_kernel(y_ref, w_ref, b_ref, o_ref):
    o_ref[0] = (jnp.dot(w_ref[...], y_ref[0], preferred_element_type=jnp.float32)
                + b_ref[...].reshape(DIM, 1))


def kernel(x, g, b_ln, W_qkv, W_out, b_out):
    b, c, h, w = x.shape
    hw = h * w
    xf = x.reshape(b, c, hw)
    gv = g.reshape(c)
    bv = b_ln.reshape(c)

    qkv = pl.pallas_call(
        _ln_qkv_kernel,
        grid=(b,),
        in_specs=[
            pl.BlockSpec((1, c, hw), lambda i: (i, 0, 0)),
            pl.BlockSpec((c,), lambda i: (0,)),
            pl.BlockSpec((c,), lambda i: (0,)),
            pl.BlockSpec((3 * INNER, c), lambda i: (0, 0)),
        ],
        out_specs=pl.BlockSpec((1, 3 * INNER, hw), lambda i: (i, 0, 0)),
        out_shape=jax.ShapeDtypeStruct((b, 3 * INNER, hw), jnp.float32),
    )(xf, gv, bv, W_qkv)

    n_tok = DIM_HEAD * h
    q = qkv[:, :INNER].reshape(b * HEADS, n_tok, w)
    k = qkv[:, INNER:2 * INNER].reshape(b * HEADS, n_tok, w)
    v = qkv[:, 2 * INNER:].reshape(b * HEADS, n_tok, w)

    o = pl.pallas_call(
        _attn_kernel,
        grid=(b * HEADS,),
        in_specs=[
            pl.BlockSpec((1, n_tok, w), lambda i: (i, 0, 0)),
            pl.BlockSpec((1, n_tok, w), lambda i: (i, 0, 0)),
            pl.BlockSpec((1, n_tok, w), lambda i: (i, 0, 0)),
        ],
        out_specs=pl.BlockSpec((1, n_tok, w), lambda i: (i, 0, 0)),
        out_shape=jax.ShapeDtypeStruct((b * HEADS, n_tok, w), jnp.float32),
    )(q, k, v)

    y = o.reshape(b, HEADS, DIM_HEAD, h, w)
    y = jnp.transpose(y, (0, 1, 3, 4, 2)).reshape(b, INNER, hw)

    out = pl.pallas_call(
        _out_proj_kernel,
        grid=(b,),
        in_specs=[
            pl.BlockSpec((1, INNER, hw), lambda i: (i, 0, 0)),
            pl.BlockSpec((DIM, INNER), lambda i: (0, 0)),
            pl.BlockSpec((DIM,), lambda i: (0,)),
        ],
        out_specs=pl.BlockSpec((1, DIM, hw), lambda i: (i, 0, 0)),
        out_shape=jax.ShapeDtypeStruct((b, DIM, hw), jnp.float32),
    )(y, W_out, b_out)

    return out.reshape(b, DIM, h, w)


# speedup vs baseline: 1.8185x; 1.3549x over previous
import jax
import jax.numpy as jnp
from jax.experimental import pallas as pl

HEADS = 8
DIM_HEAD = 32
DIM = 768
INNER = HEADS * DIM_HEAD
EPS = 1e-5


def _ln_qkv_kernel(x_ref, g_ref, bln_ref, wqkv_ref, q_ref, k_ref, v_ref):
    x = x_ref[0]
    mean = jnp.mean(x, axis=0, keepdims=True)
    var = jnp.mean((x - mean) ** 2, axis=0, keepdims=True)
    xn = (x - mean) * jax.lax.rsqrt(var + EPS)
    xn = xn * g_ref[...].reshape(DIM, 1) + bln_ref[...].reshape(DIM, 1)
    qkv = jnp.dot(wqkv_ref[...], xn, preferred_element_type=jnp.float32)
    q_ref[0] = qkv[:INNER]
    k_ref[0] = qkv[INNER:2 * INNER]
    v_ref[0] = qkv[2 * INNER:]


def _attn_kernel(q_ref, k_ref, v_ref, y_ref):
    q = q_ref[0]
    k = k_ref[0]
    v = v_ref[0]
    qn = q * jax.lax.rsqrt(jnp.maximum(jnp.sum(q * q, axis=-1, keepdims=True),
                                       1e-24))
    kn = k * jax.lax.rsqrt(jnp.maximum(jnp.sum(k * k, axis=-1, keepdims=True),
                                       1e-24))
    sim = jnp.dot(qn.astype(jnp.bfloat16), kn.astype(jnp.bfloat16).T,
                  preferred_element_type=jnp.float32)
    e = jnp.exp(sim)
    s = jnp.sum(e, axis=-1, keepdims=True)
    o = jnp.dot(e.astype(jnp.bfloat16), v.astype(jnp.bfloat16),
                preferred_element_type=jnp.float32) / s
    y = jnp.transpose(o.reshape(DIM_HEAD, 32, 32), (1, 2, 0))
    y_ref[0] = y.reshape(32, DIM_HEAD * 32)


def _out_proj_kernel(y_ref, w_ref, b_ref, o_ref):
    o_ref[0] = (jnp.dot(w_ref[...], y_ref[0], preferred_element_type=jnp.float32)
                + b_ref[...].reshape(DIM, 1))


def kernel(x, g, b_ln, W_qkv, W_out, b_out):
    b, c, h, w = x.shape
    hw = h * w
    xf = x.reshape(b, c, hw)
    gv = g.reshape(c)
    bv = b_ln.reshape(c)

    q, k, v = pl.pallas_call(
        _ln_qkv_kernel,
        grid=(b,),
        in_specs=[
            pl.BlockSpec((1, c, hw), lambda i: (i, 0, 0)),
            pl.BlockSpec((c,), lambda i: (0,)),
            pl.BlockSpec((c,), lambda i: (0,)),
            pl.BlockSpec((3 * INNER, c), lambda i: (0, 0)),
        ],
        out_specs=[
            pl.BlockSpec((1, INNER, hw), lambda i: (i, 0, 0)),
            pl.BlockSpec((1, INNER, hw), lambda i: (i, 0, 0)),
            pl.BlockSpec((1, INNER, hw), lambda i: (i, 0, 0)),
        ],
        out_shape=[
            jax.ShapeDtypeStruct((b, INNER, hw), jnp.float32),
            jax.ShapeDtypeStruct((b, INNER, hw), jnp.float32),
            jax.ShapeDtypeStruct((b, INNER, hw), jnp.float32),
        ],
    )(xf, gv, bv, W_qkv)

    n_tok = DIM_HEAD * h
    q = q.reshape(b * HEADS, n_tok, w)
    k = k.reshape(b * HEADS, n_tok, w)
    v = v.reshape(b * HEADS, n_tok, w)

    y = pl.pallas_call(
        _attn_kernel,
        grid=(b * HEADS,),
        in_specs=[
            pl.BlockSpec((1, n_tok, w), lambda i: (i, 0, 0)),
            pl.BlockSpec((1, n_tok, w), lambda i: (i, 0, 0)),
            pl.BlockSpec((1, n_tok, w), lambda i: (i, 0, 0)),
        ],
        out_specs=pl.BlockSpec((1, h, hw), lambda i: (i, 0, 0)),
        out_shape=jax.ShapeDtypeStruct((b * HEADS, h, hw), jnp.float32),
    )(q, k, v)

    y = y.reshape(b, INNER, hw)

    out = pl.pallas_call(
        _out_proj_kernel,
        grid=(b,),
        in_specs=[
            pl.BlockSpec((1, INNER, hw), lambda i: (i, 0, 0)),
            pl.BlockSpec((DIM, INNER), lambda i: (0, 0)),
            pl.BlockSpec((DIM,), lambda i: (0,)),
        ],
        out_specs=pl.BlockSpec((1, DIM, hw), lambda i: (i, 0, 0)),
        out_shape=jax.ShapeDtypeStruct((b, DIM, hw), jnp.float32),
    )(y, W_out, b_out)

    return out.reshape(b, DIM, h, w)


# speedup vs baseline: 2.2492x; 1.2368x over previous
import jax
import jax.numpy as jnp
from jax.experimental import pallas as pl

HEADS = 8
DIM_HEAD = 32
DIM = 768
INNER = HEADS * DIM_HEAD
EPS = 1e-5
ATTN_HEADS_PER_BLK = 2


def _ln_qkv_kernel(x_ref, g_ref, bln_ref, wqkv_ref, q_ref, k_ref, v_ref):
    x = x_ref[0]
    mean = jnp.mean(x, axis=0, keepdims=True)
    var = jnp.mean((x - mean) ** 2, axis=0, keepdims=True)
    xn = (x - mean) * jax.lax.rsqrt(var + EPS)
    xn = xn * g_ref[...].reshape(DIM, 1) + bln_ref[...].reshape(DIM, 1)
    qkv = jnp.dot(wqkv_ref[...], xn, preferred_element_type=jnp.float32)
    q_ref[0] = qkv[:INNER]
    k_ref[0] = qkv[INNER:2 * INNER]
    v_ref[0] = qkv[2 * INNER:]


def _attn_kernel(q_ref, k_ref, v_ref, y_ref):
    ones = jnp.ones((32, 8), dtype=jnp.float32)
    for hh in range(ATTN_HEADS_PER_BLK):
        q = q_ref[hh]
        k = k_ref[hh]
        v = v_ref[hh]
        sq = jnp.dot(q * q, ones, preferred_element_type=jnp.float32)[:, :1]
        sk = jnp.dot(k * k, ones, preferred_element_type=jnp.float32)[:, :1]
        qn = q * jax.lax.rsqrt(jnp.maximum(sq, 1e-24))
        kn = k * jax.lax.rsqrt(jnp.maximum(sk, 1e-24))
        sim = jnp.dot(qn.astype(jnp.bfloat16), kn.astype(jnp.bfloat16).T,
                      preferred_element_type=jnp.float32)
        e = jnp.exp(sim)
        s = jnp.sum(e, axis=-1, keepdims=True)
        o = jnp.dot(e.astype(jnp.bfloat16), v.astype(jnp.bfloat16),
                    preferred_element_type=jnp.float32) / s
        y = jnp.transpose(o.reshape(DIM_HEAD, 32, 32), (1, 2, 0))
        y_ref[hh] = y.reshape(32, DIM_HEAD * 32)


def _out_proj_kernel(y_ref, w_ref, b_ref, o_ref):
    o_ref[0] = (jnp.dot(w_ref[...], y_ref[0], preferred_element_type=jnp.float32)
                + b_ref[...].reshape(DIM, 1))


def kernel(x, g, b_ln, W_qkv, W_out, b_out):
    b, c, h, w = x.shape
    hw = h * w
    xf = x.reshape(b, c, hw)
    gv = g.reshape(c)
    bv = b_ln.reshape(c)

    q, k, v = pl.pallas_call(
        _ln_qkv_kernel,
        grid=(b,),
        in_specs=[
            pl.BlockSpec((1, c, hw), lambda i: (i, 0, 0)),
            pl.BlockSpec((c,), lambda i: (0,)),
            pl.BlockSpec((c,), lambda i: (0,)),
            pl.BlockSpec((3 * INNER, c), lambda i: (0, 0)),
        ],
        out_specs=[
            pl.BlockSpec((1, INNER, hw), lambda i: (i, 0, 0)),
            pl.BlockSpec((1, INNER, hw), lambda i: (i, 0, 0)),
            pl.BlockSpec((1, INNER, hw), lambda i: (i, 0, 0)),
        ],
        out_shape=[
            jax.ShapeDtypeStruct((b, INNER, hw), jnp.float32),
            jax.ShapeDtypeStruct((b, INNER, hw), jnp.float32),
            jax.ShapeDtypeStruct((b, INNER, hw), jnp.float32),
        ],
    )(xf, gv, bv, W_qkv)

    n_tok = DIM_HEAD * h
    q = q.reshape(b * HEADS, n_tok, w)
    k = k.reshape(b * HEADS, n_tok, w)
    v = v.reshape(b * HEADS, n_tok, w)

    hb = ATTN_HEADS_PER_BLK
    y = pl.pallas_call(
        _attn_kernel,
        grid=(b * HEADS // hb,),
        in_specs=[
            pl.BlockSpec((hb, n_tok, w), lambda i: (i, 0, 0)),
            pl.BlockSpec((hb, n_tok, w), lambda i: (i, 0, 0)),
            pl.BlockSpec((hb, n_tok, w), lambda i: (i, 0, 0)),
        ],
        out_specs=pl.BlockSpec((hb, h, hw), lambda i: (i, 0, 0)),
        out_shape=jax.ShapeDtypeStruct((b * HEADS, h, hw), jnp.float32),
    )(q, k, v)

    y = y.reshape(b, INNER, hw)

    out = pl.pallas_call(
        _out_proj_kernel,
        grid=(b,),
        in_specs=[
            pl.BlockSpec((1, INNER, hw), lambda i: (i, 0, 0)),
            pl.BlockSpec((DIM, INNER), lambda i: (0, 0)),
            pl.BlockSpec((DIM,), lambda i: (0,)),
        ],
        out_specs=pl.BlockSpec((1, DIM, hw), lambda i: (i, 0, 0)),
        out_shape=jax.ShapeDtypeStruct((b, DIM, hw), jnp.float32),
    )(y, W_out, b_out)

    return out.reshape(b, DIM, h, w)
